# baseline (speedup 1.0000x reference)
import functools

import jax
import jax.numpy as jnp
from jax import lax
from jax.experimental import pallas as pl
from jax.experimental.pallas import tpu as pltpu
from jax.experimental.pallas import tpu_sc as plsc

L = 16
BW = 128
NRING = 3
NBKT = 16
PSHIFT = 14


@functools.lru_cache(maxsize=None)
def _build(v, d, b):
    info = plsc.get_sparse_core_info()
    nw = info.num_cores * info.num_subcores
    nc_ = info.num_cores
    nb = -(-v // BW)
    bpt = -(-nb // nw)
    bpb = -(-bpt // NBKT)
    dump = b
    flush = 128
    pmask = (1 << PSHIFT) - 1
    assert b <= pmask + 1 and bpt * BW < (1 << (31 - PSHIFT))

    mesh = plsc.VectorSubcoreMesh(core_axis_name="c", subcore_axis_name="s")

    @functools.partial(
        pl.kernel,
        mesh=mesh,
        compiler_params=pltpu.CompilerParams(needs_layout_passes=False),
        out_type=[jax.ShapeDtypeStruct((b + 8, 2 * d), jnp.float32)],
        scratch_types=[
            pltpu.VMEM((b + L,), jnp.int32),
            pltpu.VMEM((b + L,), jnp.int32),
            pltpu.VMEM((2 * NBKT + L,), jnp.int32),
            pltpu.VMEM((NRING, d, BW), jnp.float32),
            pltpu.VMEM((NRING, d, BW), jnp.float32),
            pltpu.VMEM((BW + L,), jnp.int32),
            pltpu.VMEM((flush, 2 * d), jnp.float32),
            pltpu.VMEM((flush,), jnp.int32),
            pltpu.SemaphoreType.DMA,
            pltpu.SemaphoreType.DMA,
            pltpu.SemaphoreType.DMA,
        ],
    )
    def k(ft, mt, ranks_hbm, out_hbm,
          apk, bpk, meta, sbf, sbm, blk, rowbuf, posv,
          semf, semm, sems):
        wid = lax.axis_index("s") * nc_ + lax.axis_index("c")
        c0 = wid * bpt
        lo = c0 * BW
        hi = lo + bpt * BW
        iota = lax.iota(jnp.int32, L)
        lane0 = iota == 0

        pltpu.sync_copy(ranks_hbm, apk.at[pl.ds(0, b)])
        for g in range(flush // L):
            posv[pl.ds(g * L, L)] = jnp.full((L,), dump, jnp.int32)

        def fire(c, slot):
            cblk = c0 + c

            @pl.when((c < bpt) & (cblk < nb))
            def _():
                off = pl.multiple_of(cblk * BW, BW)
                pltpu.async_copy(ft.at[:, pl.ds(off, BW)], sbf.at[slot], semf)
                pltpu.async_copy(mt.at[:, pl.ds(off, BW)], sbm.at[slot], semm)

        def wait(c, slot):
            cblk = c0 + c

            @pl.when((c < bpt) & (cblk < nb))
            def _():
                off = pl.multiple_of(cblk * BW, BW)
                pltpu.make_async_copy(
                    ft.at[:, pl.ds(off, BW)], sbf.at[slot], semf).wait()
                pltpu.make_async_copy(
                    mt.at[:, pl.ds(off, BW)], sbm.at[slot], semm).wait()

        for c in range(NRING - 1):
            fire(c, c)

        def scan_body(g, mcount):
            vv = apk[pl.ds(g * L, L)]
            pos = jnp.full((L,), g * L, jnp.int32) + iota
            msk = (vv >= lo) & (vv < hi)
            pk = lax.shift_left(vv - lo, PSHIFT) | pos
            plsc.store_compressed(apk.at[pl.ds(mcount, L)], pk, mask=msk)
            return mcount + plsc.all_reduce_population_count(msk)[0]

        mcount = lax.fori_loop(0, b // L, scan_body, 0)

        def cnt_body(g, cnts):
            pkv = apk[pl.ds(g * L, L)]
            valid = (jnp.full((L,), g * L, jnp.int32) + iota) < mcount
            bkt = lax.shift_right_logical(pkv, PSHIFT + 7) // bpb
            return tuple(
                cnts[t] + plsc.all_reduce_population_count(
                    valid & (bkt == t))[0]
                for t in range(NBKT)
            )

        mgroups = (mcount + L - 1) // L
        cnts = lax.fori_loop(0, mgroups, cnt_body, (0,) * NBKT)
        start = 0
        for t in range(NBKT):
            plsc.store_scatter(
                meta, [jnp.full((L,), t, jnp.int32)],
                jnp.full((L,), start, jnp.int32), mask=lane0)
            plsc.store_scatter(
                meta, [jnp.full((L,), NBKT + t, jnp.int32)],
                jnp.full((L,), cnts[t], jnp.int32), mask=lane0)
            start = start + cnts[t]

        def fill_body(g, fills):
            pkv = apk[pl.ds(g * L, L)]
            valid = (jnp.full((L,), g * L, jnp.int32) + iota) < mcount
            bkt = lax.shift_right_logical(pkv, PSHIFT + 7) // bpb
            new = []
            for t in range(NBKT):
                msk = valid & (bkt == t)
                plsc.store_compressed(bpk.at[pl.ds(fills[t], L)], pkv, mask=msk)
                new.append(
                    fills[t] + plsc.all_reduce_population_count(msk)[0])
            return tuple(new)

        starts = []
        s = 0
        for t in range(NBKT):
            starts.append(s)
            s = s + cnts[t]
        lax.fori_loop(0, mgroups, fill_body, tuple(starts))


        def block_body(c, m_fill):
            slot = c % NRING
            cblk = c0 + c
            fire(c + NRING - 1, (c + NRING - 1) % NRING)
            wait(c, slot)
            slotv = jnp.full((L,), slot, jnp.int32)
            t = c // bpb
            t_start = meta[pl.ds(t, L)][0]
            t_cnt = meta[pl.ds(NBKT + t, L)][0]

            def sel_body(g, kc):
                ent0 = t_start + g * L
                pkv = bpk[pl.ds(ent0, L)]
                valid = (jnp.full((L,), g * L, jnp.int32) + iota) < t_cnt
                blkrel = lax.shift_right_logical(pkv, PSHIFT + 7)
                msk = valid & (blkrel == (cblk - c0))
                plsc.store_compressed(blk.at[pl.ds(kc, L)], pkv, mask=msk)
                return kc + plsc.all_reduce_population_count(msk)[0]

            kc = lax.fori_loop(0, (t_cnt + L - 1) // L, sel_body, 0)

            def ext_body(j, m_fill):
                pkj = blk[pl.ds(j, L)][0]
                rel = lax.shift_right_logical(pkj, PSHIFT)
                pv = pkj & pmask
                mfv = jnp.full((L,), m_fill, jnp.int32)
                colv = jnp.full((L,), rel % BW, jnp.int32)
                for k4 in range(d // L):
                    dv = iota + k4 * L
                    valf = plsc.load_gather(sbf, [slotv, dv, colv])
                    plsc.store_scatter(rowbuf, [mfv, iota + k4 * L], valf)
                    valm = plsc.load_gather(sbm, [slotv, dv, colv])
                    plsc.store_scatter(rowbuf, [mfv, iota + d + k4 * L], valm)
                plsc.store_scatter(
                    posv, [mfv], jnp.full((L,), pv, jnp.int32), mask=lane0)
                m_new = m_fill + 1

                @pl.when(m_new == flush)
                def _():
                    pltpu.async_copy(rowbuf, out_hbm.at[posv], sems).wait()
                    for g in range(flush // L):
                        posv[pl.ds(g * L, L)] = jnp.full(
                            (L,), dump, jnp.int32)

                return jnp.where(m_new == flush, 0, m_new)

            return lax.fori_loop(0, kc, ext_body, m_fill)

        m_fill = lax.fori_loop(0, bpt, block_body, 0)

        @pl.when(m_fill > 0)
        def _():
            pltpu.async_copy(rowbuf, out_hbm.at[posv], sems).wait()

    return k


def kernel(funcs, measures, ranks):
    v, d = funcs.shape
    b = ranks.shape[0]
    ft = funcs.T
    mt = measures.T
    out = _build(v, d, b)(ft, mt, ranks)[0]
    return (out[:b, :d], out[:b, d:2 * d])


# speedup vs baseline: 1.0107x; 1.0107x over previous
import functools

import jax
import jax.numpy as jnp
from jax import lax
from jax.experimental import pallas as pl
from jax.experimental.pallas import tpu as pltpu
from jax.experimental.pallas import tpu_sc as plsc

L = 16
BW = 128
NRING = 3
NBKT = 16
PSHIFT = 14


@functools.lru_cache(maxsize=None)
def _build(v, d, b):
    info = plsc.get_sparse_core_info()
    nw = info.num_cores * info.num_subcores
    nc_ = info.num_cores
    nb = -(-v // BW)
    bpt = -(-nb // nw)
    bpb = -(-bpt // NBKT)
    dump = b
    flush = 128
    pmask = (1 << PSHIFT) - 1
    assert b <= pmask + 1 and bpt * BW < (1 << (31 - PSHIFT))

    mesh = plsc.VectorSubcoreMesh(core_axis_name="c", subcore_axis_name="s")

    @functools.partial(
        pl.kernel,
        mesh=mesh,
        compiler_params=pltpu.CompilerParams(needs_layout_passes=False),
        out_type=[jax.ShapeDtypeStruct((b + 8, 2 * d), jnp.float32)],
        scratch_types=[
            pltpu.VMEM((b + L,), jnp.int32),
            pltpu.VMEM((b + L,), jnp.int32),
            pltpu.VMEM((2 * NBKT + L,), jnp.int32),
            pltpu.VMEM((NRING, d, BW), jnp.float32),
            pltpu.VMEM((NRING, d, BW), jnp.float32),
            pltpu.VMEM((BW + L,), jnp.int32),
            pltpu.VMEM((flush, 2 * d), jnp.float32),
            pltpu.VMEM((flush,), jnp.int32),
            pltpu.SemaphoreType.DMA,
            pltpu.SemaphoreType.DMA,
            pltpu.SemaphoreType.DMA,
        ],
    )
    def k(ft, mt, ranks_hbm, out_hbm,
          apk, bpk, meta, sbf, sbm, blk, rowbuf, posv,
          semf, semm, sems):
        wid = lax.axis_index("s") * nc_ + lax.axis_index("c")
        c0 = wid * bpt
        lo = c0 * BW
        hi = lo + bpt * BW
        iota = lax.iota(jnp.int32, L)
        lane0 = iota == 0

        pltpu.sync_copy(ranks_hbm, apk.at[pl.ds(0, b)])
        for g in range(flush // L):
            posv[pl.ds(g * L, L)] = jnp.full((L,), dump, jnp.int32)

        def fire(c, slot):
            cblk = c0 + c

            @pl.when((c < bpt) & (cblk < nb))
            def _():
                off = pl.multiple_of(cblk * BW, BW)
                pltpu.async_copy(ft.at[:, pl.ds(off, BW)], sbf.at[slot], semf)
                pltpu.async_copy(mt.at[:, pl.ds(off, BW)], sbm.at[slot], semm)

        def wait(c, slot):
            cblk = c0 + c

            @pl.when((c < bpt) & (cblk < nb))
            def _():
                off = pl.multiple_of(cblk * BW, BW)
                pltpu.make_async_copy(
                    ft.at[:, pl.ds(off, BW)], sbf.at[slot], semf).wait()
                pltpu.make_async_copy(
                    mt.at[:, pl.ds(off, BW)], sbm.at[slot], semm).wait()

        for c in range(NRING - 1):
            fire(c, c)

        def scan_body(g, mcount):
            vv = apk[pl.ds(g * L, L)]
            pos = jnp.full((L,), g * L, jnp.int32) + iota
            msk = (vv >= lo) & (vv < hi)
            pk = lax.shift_left(vv - lo, PSHIFT) | pos
            plsc.store_compressed(apk.at[pl.ds(mcount, L)], pk, mask=msk)
            return mcount + plsc.all_reduce_population_count(msk)[0]

        mcount = lax.fori_loop(0, b // L, scan_body, 0)

        def cnt_body(g, cnts):
            pkv = apk[pl.ds(g * L, L)]
            valid = (jnp.full((L,), g * L, jnp.int32) + iota) < mcount
            bkt = lax.shift_right_logical(pkv, PSHIFT + 7) // bpb
            return tuple(
                cnts[t] + plsc.all_reduce_population_count(
                    valid & (bkt == t))[0]
                for t in range(NBKT)
            )

        mgroups = (mcount + L - 1) // L
        cnts = lax.fori_loop(0, mgroups, cnt_body, (0,) * NBKT)
        start = 0
        for t in range(NBKT):
            plsc.store_scatter(
                meta, [jnp.full((L,), t, jnp.int32)],
                jnp.full((L,), start, jnp.int32), mask=lane0)
            plsc.store_scatter(
                meta, [jnp.full((L,), NBKT + t, jnp.int32)],
                jnp.full((L,), cnts[t], jnp.int32), mask=lane0)
            start = start + cnts[t]

        def fill_body(g, fills):
            pkv = apk[pl.ds(g * L, L)]
            valid = (jnp.full((L,), g * L, jnp.int32) + iota) < mcount
            bkt = lax.shift_right_logical(pkv, PSHIFT + 7) // bpb
            new = []
            for t in range(NBKT):
                msk = valid & (bkt == t)
                plsc.store_compressed(bpk.at[pl.ds(fills[t], L)], pkv, mask=msk)
                new.append(
                    fills[t] + plsc.all_reduce_population_count(msk)[0])
            return tuple(new)

        starts = []
        s = 0
        for t in range(NBKT):
            starts.append(s)
            s = s + cnts[t]
        lax.fori_loop(0, mgroups, fill_body, tuple(starts))


        def block_body(c, m_fill):
            slot = c % NRING
            cblk = c0 + c
            fire(c + NRING - 1, (c + NRING - 1) % NRING)
            slotv = jnp.full((L,), slot, jnp.int32)
            t = c // bpb
            t_start = meta[pl.ds(t, L)][0]
            t_cnt = meta[pl.ds(NBKT + t, L)][0]

            def sel_body(g, kc):
                ent0 = t_start + g * L
                pkv = bpk[pl.ds(ent0, L)]
                valid = (jnp.full((L,), g * L, jnp.int32) + iota) < t_cnt
                blkrel = lax.shift_right_logical(pkv, PSHIFT + 7)
                msk = valid & (blkrel == (cblk - c0))
                plsc.store_compressed(blk.at[pl.ds(kc, L)], pkv, mask=msk)
                return kc + plsc.all_reduce_population_count(msk)[0]

            kc = lax.fori_loop(0, (t_cnt + L - 1) // L, sel_body, 0)
            wait(c, slot)

            def ext_body(j, m_fill):
                pkj = blk[pl.ds(j, L)][0]
                rel = lax.shift_right_logical(pkj, PSHIFT)
                pv = pkj & pmask
                mfv = jnp.full((L,), m_fill, jnp.int32)
                colv = jnp.full((L,), rel % BW, jnp.int32)
                for k4 in range(d // L):
                    dv = iota + k4 * L
                    valf = plsc.load_gather(sbf, [slotv, dv, colv])
                    plsc.store_scatter(rowbuf, [mfv, iota + k4 * L], valf)
                    valm = plsc.load_gather(sbm, [slotv, dv, colv])
                    plsc.store_scatter(rowbuf, [mfv, iota + d + k4 * L], valm)
                plsc.store_scatter(
                    posv, [mfv], jnp.full((L,), pv, jnp.int32), mask=lane0)
                m_new = m_fill + 1

                @pl.when(m_new == flush)
                def _():
                    pltpu.async_copy(rowbuf, out_hbm.at[posv], sems).wait()
                    for g in range(flush // L):
                        posv[pl.ds(g * L, L)] = jnp.full(
                            (L,), dump, jnp.int32)

                return jnp.where(m_new == flush, 0, m_new)

            return lax.fori_loop(0, kc, ext_body, m_fill)

        m_fill = lax.fori_loop(0, bpt, block_body, 0)

        @pl.when(m_fill > 0)
        def _():
            pltpu.async_copy(rowbuf, out_hbm.at[posv], sems).wait()

    return k


def kernel(funcs, measures, ranks):
    v, d = funcs.shape
    b = ranks.shape[0]
    ft = funcs.T
    mt = measures.T
    out = _build(v, d, b)(ft, mt, ranks)[0]
    return (out[:b, :d], out[:b, d:2 * d])


# speedup vs baseline: 1.1595x; 1.1472x over previous
import functools

import jax
import jax.numpy as jnp
from jax import lax
from jax.experimental import pallas as pl
from jax.experimental.pallas import tpu as pltpu
from jax.experimental.pallas import tpu_sc as plsc

L = 16
BW = 128
NRING = 4
NBKT = 16
PSHIFT = 14


@functools.lru_cache(maxsize=None)
def _build(v, d, b):
    info = plsc.get_sparse_core_info()
    nw = info.num_cores * info.num_subcores
    nc_ = info.num_cores
    nb = -(-v // BW)
    bpt = -(-nb // nw)
    bpb = -(-bpt // NBKT)
    dump = b
    flush = 128
    pmask = (1 << PSHIFT) - 1
    assert b <= pmask + 1 and bpt * BW < (1 << (31 - PSHIFT))

    mesh = plsc.VectorSubcoreMesh(core_axis_name="c", subcore_axis_name="s")

    @functools.partial(
        pl.kernel,
        mesh=mesh,
        compiler_params=pltpu.CompilerParams(needs_layout_passes=False),
        out_type=[jax.ShapeDtypeStruct((b + 8, 2 * d), jnp.float32)],
        scratch_types=[
            pltpu.VMEM((b + L,), jnp.int32),
            pltpu.VMEM((b + L,), jnp.int32),
            pltpu.VMEM((2 * NBKT + L,), jnp.int32),
            pltpu.VMEM((NRING, d, BW), jnp.float32),
            pltpu.VMEM((NRING, d, BW), jnp.float32),
            pltpu.VMEM((BW + L,), jnp.int32),
            pltpu.VMEM((flush, 2 * d), jnp.float32),
            pltpu.VMEM((flush,), jnp.int32),
            pltpu.SemaphoreType.DMA,
            pltpu.SemaphoreType.DMA,
            pltpu.SemaphoreType.DMA,
        ],
    )
    def k(ft, mt, ranks_hbm, out_hbm,
          apk, bpk, meta, sbf, sbm, blk, rowbuf, posv,
          semf, semm, sems):
        wid = lax.axis_index("s") * nc_ + lax.axis_index("c")
        c0 = wid * bpt
        lo = c0 * BW
        hi = lo + bpt * BW
        iota = lax.iota(jnp.int32, L)
        lane0 = iota == 0

        pltpu.sync_copy(ranks_hbm, apk.at[pl.ds(0, b)])
        for g in range(flush // L):
            posv[pl.ds(g * L, L)] = jnp.full((L,), dump, jnp.int32)

        def fire(c, slot):
            cblk = c0 + c

            @pl.when((c < bpt) & (cblk < nb))
            def _():
                off = pl.multiple_of(cblk * BW, BW)
                pltpu.async_copy(ft.at[:, pl.ds(off, BW)], sbf.at[slot], semf)
                pltpu.async_copy(mt.at[:, pl.ds(off, BW)], sbm.at[slot], semm)

        def wait(c, slot):
            cblk = c0 + c

            @pl.when((c < bpt) & (cblk < nb))
            def _():
                off = pl.multiple_of(cblk * BW, BW)
                pltpu.make_async_copy(
                    ft.at[:, pl.ds(off, BW)], sbf.at[slot], semf).wait()
                pltpu.make_async_copy(
                    mt.at[:, pl.ds(off, BW)], sbm.at[slot], semm).wait()

        for c in range(NRING - 1):
            fire(c, c)

        def scan_body(g, mcount):
            vv = apk[pl.ds(g * L, L)]
            pos = jnp.full((L,), g * L, jnp.int32) + iota
            msk = (vv >= lo) & (vv < hi)
            pk = lax.shift_left(vv - lo, PSHIFT) | pos
            plsc.store_compressed(apk.at[pl.ds(mcount, L)], pk, mask=msk)
            return mcount + plsc.all_reduce_population_count(msk)[0]

        mcount = lax.fori_loop(0, b // L, scan_body, 0)

        def cnt_body(g, cnts):
            pkv = apk[pl.ds(g * L, L)]
            valid = (jnp.full((L,), g * L, jnp.int32) + iota) < mcount
            bkt = lax.shift_right_logical(pkv, PSHIFT + 7) // bpb
            return tuple(
                cnts[t] + plsc.all_reduce_population_count(
                    valid & (bkt == t))[0]
                for t in range(NBKT)
            )

        mgroups = (mcount + L - 1) // L
        cnts = lax.fori_loop(0, mgroups, cnt_body, (0,) * NBKT)
        start = 0
        for t in range(NBKT):
            plsc.store_scatter(
                meta, [jnp.full((L,), t, jnp.int32)],
                jnp.full((L,), start, jnp.int32), mask=lane0)
            plsc.store_scatter(
                meta, [jnp.full((L,), NBKT + t, jnp.int32)],
                jnp.full((L,), cnts[t], jnp.int32), mask=lane0)
            start = start + cnts[t]

        def fill_body(g, fills):
            pkv = apk[pl.ds(g * L, L)]
            valid = (jnp.full((L,), g * L, jnp.int32) + iota) < mcount
            bkt = lax.shift_right_logical(pkv, PSHIFT + 7) // bpb
            new = []
            for t in range(NBKT):
                msk = valid & (bkt == t)
                plsc.store_compressed(bpk.at[pl.ds(fills[t], L)], pkv, mask=msk)
                new.append(
                    fills[t] + plsc.all_reduce_population_count(msk)[0])
            return tuple(new)

        starts = []
        s = 0
        for t in range(NBKT):
            starts.append(s)
            s = s + cnts[t]
        lax.fori_loop(0, mgroups, fill_body, tuple(starts))


        def block_body(c, m_fill):
            slot = c % NRING
            cblk = c0 + c
            fire(c + NRING - 1, (c + NRING - 1) % NRING)
            slotv = jnp.full((L,), slot, jnp.int32)
            t = c // bpb
            t_start = meta[pl.ds(t, L)][0]
            t_cnt = meta[pl.ds(NBKT + t, L)][0]

            def sel_body(g, kc):
                ent0 = t_start + g * L
                pkv = bpk[pl.ds(ent0, L)]
                valid = (jnp.full((L,), g * L, jnp.int32) + iota) < t_cnt
                blkrel = lax.shift_right_logical(pkv, PSHIFT + 7)
                msk = valid & (blkrel == (cblk - c0))
                plsc.store_compressed(blk.at[pl.ds(kc, L)], pkv, mask=msk)
                return kc + plsc.all_reduce_population_count(msk)[0]

            kc = lax.fori_loop(0, (t_cnt + L - 1) // L, sel_body, 0)
            wait(c, slot)

            def ext_body(j, m_fill):
                pkj = blk[pl.ds(j, L)][0]
                rel = lax.shift_right_logical(pkj, PSHIFT)
                pv = pkj & pmask
                mfv = jnp.full((L,), m_fill, jnp.int32)
                colv = jnp.full((L,), rel % BW, jnp.int32)
                for k4 in range(d // L):
                    dv = iota + k4 * L
                    valf = plsc.load_gather(sbf, [slotv, dv, colv])
                    plsc.store_scatter(rowbuf, [mfv, iota + k4 * L], valf)
                    valm = plsc.load_gather(sbm, [slotv, dv, colv])
                    plsc.store_scatter(rowbuf, [mfv, iota + d + k4 * L], valm)
                plsc.store_scatter(
                    posv, [mfv], jnp.full((L,), pv, jnp.int32), mask=lane0)
                m_new = m_fill + 1

                @pl.when(m_new == flush)
                def _():
                    pltpu.async_copy(rowbuf, out_hbm.at[posv], sems).wait()
                    for g in range(flush // L):
                        posv[pl.ds(g * L, L)] = jnp.full(
                            (L,), dump, jnp.int32)

                return jnp.where(m_new == flush, 0, m_new)

            return lax.fori_loop(0, kc, ext_body, m_fill)

        m_fill = lax.fori_loop(0, bpt, block_body, 0)

        @pl.when(m_fill > 0)
        def _():
            pltpu.async_copy(rowbuf, out_hbm.at[posv], sems).wait()

    return k


def kernel(funcs, measures, ranks):
    v, d = funcs.shape
    b = ranks.shape[0]
    ft = funcs.T
    mt = measures.T
    out = _build(v, d, b)(ft, mt, ranks)[0]
    return (out[:b, :d], out[:b, d:2 * d])


# speedup vs baseline: 1.2838x; 1.1072x over previous
import functools

import jax
import jax.numpy as jnp
from jax import lax
from jax.experimental import pallas as pl
from jax.experimental.pallas import tpu as pltpu
from jax.experimental.pallas import tpu_sc as plsc

L = 16
BW = 128
NRING = 5
NBKT = 16
PSHIFT = 14


@functools.lru_cache(maxsize=None)
def _build(v, d, b):
    info = plsc.get_sparse_core_info()
    nw = info.num_cores * info.num_subcores
    nc_ = info.num_cores
    nb = -(-v // BW)
    bpt = -(-nb // nw)
    bpb = -(-bpt // NBKT)
    dump = b
    flush = 64
    pmask = (1 << PSHIFT) - 1
    assert b <= pmask + 1 and bpt * BW < (1 << (31 - PSHIFT))

    mesh = plsc.VectorSubcoreMesh(core_axis_name="c", subcore_axis_name="s")

    @functools.partial(
        pl.kernel,
        mesh=mesh,
        compiler_params=pltpu.CompilerParams(needs_layout_passes=False),
        out_type=[jax.ShapeDtypeStruct((b + 8, 2 * d), jnp.float32)],
        scratch_types=[
            pltpu.VMEM((b + L,), jnp.int32),
            pltpu.VMEM((b + L,), jnp.int32),
            pltpu.VMEM((2 * NBKT + L,), jnp.int32),
            pltpu.VMEM((NRING, d, BW), jnp.float32),
            pltpu.VMEM((NRING, d, BW), jnp.float32),
            pltpu.VMEM((BW + L,), jnp.int32),
            pltpu.VMEM((flush, 2 * d), jnp.float32),
            pltpu.VMEM((flush,), jnp.int32),
            pltpu.SemaphoreType.DMA,
            pltpu.SemaphoreType.DMA,
            pltpu.SemaphoreType.DMA,
        ],
    )
    def k(ft, mt, ranks_hbm, out_hbm,
          apk, bpk, meta, sbf, sbm, blk, rowbuf, posv,
          semf, semm, sems):
        wid = lax.axis_index("s") * nc_ + lax.axis_index("c")
        c0 = wid * bpt
        lo = c0 * BW
        hi = lo + bpt * BW
        iota = lax.iota(jnp.int32, L)
        lane0 = iota == 0

        pltpu.sync_copy(ranks_hbm, apk.at[pl.ds(0, b)])
        for g in range(flush // L):
            posv[pl.ds(g * L, L)] = jnp.full((L,), dump, jnp.int32)

        def fire(c, slot):
            cblk = c0 + c

            @pl.when((c < bpt) & (cblk < nb))
            def _():
                off = pl.multiple_of(cblk * BW, BW)
                pltpu.async_copy(ft.at[:, pl.ds(off, BW)], sbf.at[slot], semf)
                pltpu.async_copy(mt.at[:, pl.ds(off, BW)], sbm.at[slot], semm)

        def wait(c, slot):
            cblk = c0 + c

            @pl.when((c < bpt) & (cblk < nb))
            def _():
                off = pl.multiple_of(cblk * BW, BW)
                pltpu.make_async_copy(
                    ft.at[:, pl.ds(off, BW)], sbf.at[slot], semf).wait()
                pltpu.make_async_copy(
                    mt.at[:, pl.ds(off, BW)], sbm.at[slot], semm).wait()

        for c in range(NRING - 1):
            fire(c, c)

        def scan_body(g, mcount):
            vv = apk[pl.ds(g * L, L)]
            pos = jnp.full((L,), g * L, jnp.int32) + iota
            msk = (vv >= lo) & (vv < hi)
            pk = lax.shift_left(vv - lo, PSHIFT) | pos
            plsc.store_compressed(apk.at[pl.ds(mcount, L)], pk, mask=msk)
            return mcount + plsc.all_reduce_population_count(msk)[0]

        mcount = lax.fori_loop(0, b // L, scan_body, 0)

        def cnt_body(g, cnts):
            pkv = apk[pl.ds(g * L, L)]
            valid = (jnp.full((L,), g * L, jnp.int32) + iota) < mcount
            bkt = lax.shift_right_logical(pkv, PSHIFT + 7) // bpb
            return tuple(
                cnts[t] + plsc.all_reduce_population_count(
                    valid & (bkt == t))[0]
                for t in range(NBKT)
            )

        mgroups = (mcount + L - 1) // L
        cnts = lax.fori_loop(0, mgroups, cnt_body, (0,) * NBKT)
        start = 0
        for t in range(NBKT):
            plsc.store_scatter(
                meta, [jnp.full((L,), t, jnp.int32)],
                jnp.full((L,), start, jnp.int32), mask=lane0)
            plsc.store_scatter(
                meta, [jnp.full((L,), NBKT + t, jnp.int32)],
                jnp.full((L,), cnts[t], jnp.int32), mask=lane0)
            start = start + cnts[t]

        def fill_body(g, fills):
            pkv = apk[pl.ds(g * L, L)]
            valid = (jnp.full((L,), g * L, jnp.int32) + iota) < mcount
            bkt = lax.shift_right_logical(pkv, PSHIFT + 7) // bpb
            new = []
            for t in range(NBKT):
                msk = valid & (bkt == t)
                plsc.store_compressed(bpk.at[pl.ds(fills[t], L)], pkv, mask=msk)
                new.append(
                    fills[t] + plsc.all_reduce_population_count(msk)[0])
            return tuple(new)

        starts = []
        s = 0
        for t in range(NBKT):
            starts.append(s)
            s = s + cnts[t]
        lax.fori_loop(0, mgroups, fill_body, tuple(starts))


        def block_body(c, m_fill):
            slot = c % NRING
            cblk = c0 + c
            fire(c + NRING - 1, (c + NRING - 1) % NRING)
            slotv = jnp.full((L,), slot, jnp.int32)
            t = c // bpb
            t_start = meta[pl.ds(t, L)][0]
            t_cnt = meta[pl.ds(NBKT + t, L)][0]

            def sel_body(g, kc):
                ent0 = t_start + g * L
                pkv = bpk[pl.ds(ent0, L)]
                valid = (jnp.full((L,), g * L, jnp.int32) + iota) < t_cnt
                blkrel = lax.shift_right_logical(pkv, PSHIFT + 7)
                msk = valid & (blkrel == (cblk - c0))
                plsc.store_compressed(blk.at[pl.ds(kc, L)], pkv, mask=msk)
                return kc + plsc.all_reduce_population_count(msk)[0]

            kc = lax.fori_loop(0, (t_cnt + L - 1) // L, sel_body, 0)
            wait(c, slot)

            def ext_body(j, m_fill):
                pkj = blk[pl.ds(j, L)][0]
                rel = lax.shift_right_logical(pkj, PSHIFT)
                pv = pkj & pmask
                mfv = jnp.full((L,), m_fill, jnp.int32)
                colv = jnp.full((L,), rel % BW, jnp.int32)
                for k4 in range(d // L):
                    dv = iota + k4 * L
                    valf = plsc.load_gather(sbf, [slotv, dv, colv])
                    plsc.store_scatter(rowbuf, [mfv, iota + k4 * L], valf)
                    valm = plsc.load_gather(sbm, [slotv, dv, colv])
                    plsc.store_scatter(rowbuf, [mfv, iota + d + k4 * L], valm)
                plsc.store_scatter(
                    posv, [mfv], jnp.full((L,), pv, jnp.int32), mask=lane0)
                m_new = m_fill + 1

                @pl.when(m_new == flush)
                def _():
                    pltpu.async_copy(rowbuf, out_hbm.at[posv], sems).wait()
                    for g in range(flush // L):
                        posv[pl.ds(g * L, L)] = jnp.full(
                            (L,), dump, jnp.int32)

                return jnp.where(m_new == flush, 0, m_new)

            return lax.fori_loop(0, kc, ext_body, m_fill)

        m_fill = lax.fori_loop(0, bpt, block_body, 0)

        @pl.when(m_fill > 0)
        def _():
            pltpu.async_copy(rowbuf, out_hbm.at[posv], sems).wait()

    return k


def kernel(funcs, measures, ranks):
    v, d = funcs.shape
    b = ranks.shape[0]
    ft = funcs.T
    mt = measures.T
    out = _build(v, d, b)(ft, mt, ranks)[0]
    return (out[:b, :d], out[:b, d:2 * d])
